# Initial kernel scaffold; baseline (speedup 1.0000x reference)
import functools
import math

import jax
import jax.numpy as jnp
from jax import lax
from jax.experimental import pallas as pl
from jax.experimental.pallas import tpu as pltpu
from jax.experimental.pallas import tpu_sc as plsc

_B = 16384
_D = 128
_C = 100
_NB = 64
_CPAD = 112
_L = 16
_NC, _NS = 2, 16
_NW = _NC * _NS
_BPW = _B // _NW
_SUB = 256
_NSUB = _BPW // _SUB
_NGRP = _SUB // _L
_NEG_HALF = jnp.float32(-0.5)
_K = jnp.float32(-_D * math.log(2.0 * math.pi))


def _body(z0f_hbm, zTf_hbm, meansf_hbm, lvf_hbm, logc_hbm, logb_hbm,
          ct_hbm, bt_hbm, out_hbm,
          means_v, lv_v, logc_v, logb_v, ct_v, bt_v, z0_v, zT_v, out_v):
    wid = lax.axis_index("s") * _NC + lax.axis_index("c")
    base = wid * _BPW

    pltpu.sync_copy(meansf_hbm, means_v)
    pltpu.sync_copy(lvf_hbm, lv_v)
    pltpu.sync_copy(logc_hbm, logc_v)
    pltpu.sync_copy(logb_hbm, logb_v)
    pltpu.sync_copy(ct_hbm.at[pl.ds(base, _BPW)], ct_v)
    pltpu.sync_copy(bt_hbm.at[pl.ds(base, _BPW)], bt_v)

    lane = lax.iota(jnp.int32, _L)

    for s in range(_NSUB):
        off = base * _D + s * _SUB * _D
        pltpu.sync_copy(z0f_hbm.at[pl.ds(off, _SUB * _D)], z0_v)
        pltpu.sync_copy(zTf_hbm.at[pl.ds(off, _SUB * _D)], zT_v)
        for g in range(_NGRP):
            lo = s * _SUB + g * _L
            ct = ct_v[pl.ds(lo, _L)]
            bt = bt_v[pl.ds(lo, _L)]
            cat = (plsc.load_gather(logc_v, [ct])
                   + plsc.load_gather(logb_v, [bt]))
            pv0 = (jnp.int32(g * _L) + lane) * _D
            zero = jnp.zeros((_L,), jnp.float32)

            def dbody(_, carry):
                pv, mi, q, t = carry
                z0 = plsc.load_gather(z0_v, [pv])
                zT = plsc.load_gather(zT_v, [pv])
                mu = plsc.load_gather(means_v, [mi])
                lv = plsc.load_gather(lv_v, [mi])
                dlt = z0 - mu
                q = q + (dlt * dlt * jnp.exp(-lv) + lv)
                t = t + zT * zT
                return (pv + 1, mi + _C, q, t)

            _, _, q, t = lax.fori_loop(0, _D, dbody, (pv0, ct, zero, zero))
            out_v[pl.ds(lo, _L)] = _NEG_HALF * (q + t) + cat + _K

    pltpu.sync_copy(out_v, out_hbm.at[pl.ds(base, _BPW)])


_sc_call = pl.kernel(
    _body,
    out_type=jax.ShapeDtypeStruct((_B,), jnp.float32),
    mesh=plsc.VectorSubcoreMesh(core_axis_name="c", subcore_axis_name="s"),
    scratch_types=[
        pltpu.VMEM((_D * _C,), jnp.float32),
        pltpu.VMEM((_D * _C,), jnp.float32),
        pltpu.VMEM((_CPAD,), jnp.float32),
        pltpu.VMEM((_NB,), jnp.float32),
        pltpu.VMEM((_BPW,), jnp.int32),
        pltpu.VMEM((_BPW,), jnp.int32),
        pltpu.VMEM((_SUB * _D,), jnp.float32),
        pltpu.VMEM((_SUB * _D,), jnp.float32),
        pltpu.VMEM((_BPW,), jnp.float32),
    ],
)


def kernel(z0, zT, means, log_vars, cell_probs, batch_probs,
           cell_type, batch_idx):
    z0f = z0.reshape(-1)
    zTf = zT.reshape(-1)
    meansf = means.T.reshape(-1)
    lvf = log_vars.T.reshape(-1)
    logc = jnp.log(jnp.concatenate(
        [cell_probs, jnp.ones((_CPAD - _C,), jnp.float32)]))
    logb = jnp.log(batch_probs)
    ct = cell_type.astype(jnp.int32)
    bt = batch_idx.astype(jnp.int32)
    return _sc_call(z0f, zTf, meansf, lvf, logc, logb, ct, bt)


# speedup vs baseline: 2.8130x; 2.8130x over previous
import functools
import math

import jax
import jax.numpy as jnp
from jax import lax
from jax.experimental import pallas as pl
from jax.experimental.pallas import tpu as pltpu
from jax.experimental.pallas import tpu_sc as plsc

_B = 16384
_D = 128
_C = 100
_NB = 64
_CPAD = 112
_L = 16
_NC, _NS = 2, 16
_NW = _NC * _NS
_BPW = _B // _NW
_SUB = 256
_NSUB = _BPW // _SUB
_NGRP = _SUB // _L
_K = -_D * math.log(2.0 * math.pi)


def _body(z0f_hbm, zTf_hbm, meansf_hbm, lvf_hbm, logc_hbm, logb_hbm,
          ct_hbm, bt_hbm, out_hbm,
          means_v, lv_v, logc_v, logb_v, ct_v, bt_v, z0_v, zT_v, out_v):
    wid = lax.axis_index("s") * _NC + lax.axis_index("c")
    base = wid * _BPW

    pltpu.sync_copy(meansf_hbm, means_v)
    pltpu.sync_copy(lvf_hbm, lv_v)
    pltpu.sync_copy(logc_hbm, logc_v)
    pltpu.sync_copy(logb_hbm, logb_v)
    pltpu.sync_copy(ct_hbm.at[pl.ds(base, _BPW)], ct_v)
    pltpu.sync_copy(bt_hbm.at[pl.ds(base, _BPW)], bt_v)

    lane = lax.iota(jnp.int32, _L)

    for s in range(_NSUB):
        off = base * _D + s * _SUB * _D
        pltpu.sync_copy(z0f_hbm.at[pl.ds(off, _SUB * _D)], z0_v)
        pltpu.sync_copy(zTf_hbm.at[pl.ds(off, _SUB * _D)], zT_v)
        for g in range(_NGRP):
            lo = s * _SUB + g * _L
            ct = ct_v[pl.ds(lo, _L)]
            bt = bt_v[pl.ds(lo, _L)]
            cat = (plsc.load_gather(logc_v, [ct])
                   + plsc.load_gather(logb_v, [bt]))
            pv0 = (jnp.int32(g * _L) + lane) * _D
            zero = jnp.zeros((_L,), jnp.float32)

            def dbody(_, carry):
                pv, mi, q, t = carry
                z0 = plsc.load_gather(z0_v, [pv])
                zT = plsc.load_gather(zT_v, [pv])
                mu = plsc.load_gather(means_v, [mi])
                lv = plsc.load_gather(lv_v, [mi])
                dlt = z0 - mu
                q = q + (dlt * dlt * jnp.exp(-lv) + lv)
                t = t + zT * zT
                return (pv + 1, mi + _C, q, t)

            _, _, q, t = lax.fori_loop(0, _D, dbody, (pv0, ct, zero, zero))
            out_v[pl.ds(lo, _L)] = -0.5 * (q + t) + (cat + jnp.float32(_K))

    pltpu.sync_copy(out_v, out_hbm.at[pl.ds(base, _BPW)])


_sc_call = pl.kernel(
    _body,
    out_type=jax.ShapeDtypeStruct((_B,), jnp.float32),
    mesh=plsc.VectorSubcoreMesh(core_axis_name="c", subcore_axis_name="s"),
    compiler_params=pltpu.CompilerParams(needs_layout_passes=False),
    scratch_types=[
        pltpu.VMEM((_D * _C,), jnp.float32),
        pltpu.VMEM((_D * _C,), jnp.float32),
        pltpu.VMEM((_CPAD,), jnp.float32),
        pltpu.VMEM((_NB,), jnp.float32),
        pltpu.VMEM((_BPW,), jnp.int32),
        pltpu.VMEM((_BPW,), jnp.int32),
        pltpu.VMEM((_SUB * _D,), jnp.float32),
        pltpu.VMEM((_SUB * _D,), jnp.float32),
        pltpu.VMEM((_BPW,), jnp.float32),
    ],
)


def kernel(z0, zT, means, log_vars, cell_probs, batch_probs,
           cell_type, batch_idx):
    z0f = z0.reshape(-1)
    zTf = zT.reshape(-1)
    meansf = means.T.reshape(-1)
    lvf = log_vars.T.reshape(-1)
    logc = jnp.log(jnp.concatenate(
        [cell_probs, jnp.ones((_CPAD - _C,), jnp.float32)]))
    logb = jnp.log(batch_probs)
    ct = cell_type.astype(jnp.int32)
    bt = batch_idx.astype(jnp.int32)
    return _sc_call(z0f, zTf, meansf, lvf, logc, logb, ct, bt)


# speedup vs baseline: 2.9781x; 1.0587x over previous
import functools
import math

import jax
import jax.numpy as jnp
from jax import lax
from jax.experimental import pallas as pl
from jax.experimental.pallas import tpu as pltpu
from jax.experimental.pallas import tpu_sc as plsc

_B = 16384
_D = 128
_C = 100
_NB = 64
_CP = 112
_L = 16
_NC, _NS = 2, 16
_NW = _NC * _NS
_BPW = _B // _NW
_SUB = 256
_NSUB = _BPW // _SUB
_NGRP = _SUB // _L
_UNROLL = 4
_K = -_D * math.log(2.0 * math.pi)


def _body(z0f_hbm, zTf_hbm, meansf_hbm, lvf_hbm, logc_hbm, logb_hbm,
          ct_hbm, bt_hbm, out_hbm,
          means_v, iv_v, a_v, logb_v, ct_v, bt_v, z0_v, zT_v, out_v):
    wid = lax.axis_index("s") * _NC + lax.axis_index("c")
    base = wid * _BPW

    pltpu.sync_copy(meansf_hbm, means_v)
    pltpu.sync_copy(lvf_hbm, iv_v)
    pltpu.sync_copy(logc_hbm, a_v)
    pltpu.sync_copy(logb_hbm, logb_v)
    pltpu.sync_copy(ct_hbm.at[pl.ds(base, _BPW)], ct_v)
    pltpu.sync_copy(bt_hbm.at[pl.ds(base, _BPW)], bt_v)

    nk = _CP // _L
    zero = jnp.zeros((_L,), jnp.float32)

    def pre_body(d, accs):
        out = []
        for k in range(nk):
            v = iv_v[pl.ds(d * _CP + k * _L, _L)]
            iv_v[pl.ds(d * _CP + k * _L, _L)] = jnp.exp(-v)
            out.append(accs[k] + v)
        return tuple(out)

    sums = lax.fori_loop(0, _D, pre_body, (zero,) * nk)
    for k in range(nk):
        a_v[pl.ds(k * _L, _L)] = a_v[pl.ds(k * _L, _L)] - 0.5 * sums[k]

    lane = lax.iota(jnp.int32, _L)

    for s in range(_NSUB):
        off = base * _D + s * _SUB * _D
        pltpu.sync_copy(z0f_hbm.at[pl.ds(off, _SUB * _D)], z0_v)
        pltpu.sync_copy(zTf_hbm.at[pl.ds(off, _SUB * _D)], zT_v)
        for g in range(_NGRP):
            lo = s * _SUB + g * _L
            ct = ct_v[pl.ds(lo, _L)]
            bt = bt_v[pl.ds(lo, _L)]
            cat = (plsc.load_gather(a_v, [ct])
                   + plsc.load_gather(logb_v, [bt]))
            pv0 = (jnp.int32(g * _L) + lane) * _D

            def dbody(_, carry):
                pv, mi, q0, q1, t0, t1 = carry
                qs = [q0, q1]
                ts = [t0, t1]
                for u in range(_UNROLL):
                    pvu = pv + u if u else pv
                    miu = mi + u * _CP if u else mi
                    z0 = plsc.load_gather(z0_v, [pvu])
                    zT = plsc.load_gather(zT_v, [pvu])
                    mu = plsc.load_gather(means_v, [miu])
                    iv = plsc.load_gather(iv_v, [miu])
                    dlt = z0 - mu
                    qs[u % 2] = qs[u % 2] + dlt * dlt * iv
                    ts[u % 2] = ts[u % 2] + zT * zT
                return (pv + _UNROLL, mi + _UNROLL * _CP,
                        qs[0], qs[1], ts[0], ts[1])

            _, _, q0, q1, t0, t1 = lax.fori_loop(
                0, _D // _UNROLL, dbody,
                (pv0, ct, zero, zero, zero, zero))
            out_v[pl.ds(lo, _L)] = (-0.5 * ((q0 + q1) + (t0 + t1))
                                    + (cat + jnp.float32(_K)))

    pltpu.sync_copy(out_v, out_hbm.at[pl.ds(base, _BPW)])


_sc_call = pl.kernel(
    _body,
    out_type=jax.ShapeDtypeStruct((_B,), jnp.float32),
    mesh=plsc.VectorSubcoreMesh(core_axis_name="c", subcore_axis_name="s"),
    compiler_params=pltpu.CompilerParams(needs_layout_passes=False),
    scratch_types=[
        pltpu.VMEM((_D * _CP,), jnp.float32),
        pltpu.VMEM((_D * _CP,), jnp.float32),
        pltpu.VMEM((_CP,), jnp.float32),
        pltpu.VMEM((_NB,), jnp.float32),
        pltpu.VMEM((_BPW,), jnp.int32),
        pltpu.VMEM((_BPW,), jnp.int32),
        pltpu.VMEM((_SUB * _D,), jnp.float32),
        pltpu.VMEM((_SUB * _D,), jnp.float32),
        pltpu.VMEM((_BPW,), jnp.float32),
    ],
)


def kernel(z0, zT, means, log_vars, cell_probs, batch_probs,
           cell_type, batch_idx):
    z0f = z0.reshape(-1)
    zTf = zT.reshape(-1)
    pad = jnp.zeros((_CP - _C, _D), jnp.float32)
    meansf = jnp.concatenate([means, pad]).T.reshape(-1)
    lvf = jnp.concatenate([log_vars, pad]).T.reshape(-1)
    logc = jnp.log(jnp.concatenate(
        [cell_probs, jnp.ones((_CP - _C,), jnp.float32)]))
    logb = jnp.log(batch_probs)
    ct = cell_type.astype(jnp.int32)
    bt = batch_idx.astype(jnp.int32)
    return _sc_call(z0f, zTf, meansf, lvf, logc, logb, ct, bt)


# speedup vs baseline: 5.4435x; 1.8278x over previous
import functools
import math

import jax
import jax.numpy as jnp
from jax import lax
from jax.experimental import pallas as pl
from jax.experimental.pallas import tpu as pltpu
from jax.experimental.pallas import tpu_sc as plsc

_B = 16384
_D = 128
_C = 100
_NB = 64
_CP = 112
_L = 16
_NC, _NS = 2, 16
_NW = _NC * _NS
_BPW = _B // _NW
_SUB = 256
_NSUB = _BPW // _SUB
_NGRP = _SUB // _L
_UNROLL = 4
_K = -_D * math.log(2.0 * math.pi)


def _body(z0f_hbm, zTf_hbm, meansf_hbm, lvf_hbm, lvtf_hbm, logc_hbm,
          logb_hbm, ct_hbm, bt_hbm, out_hbm,
          means_v, iv_v, lvt_v, a_v, logb_v, ct_v, bt_v, z0_v, zT_v, out_v):
    wid = lax.axis_index("s") * _NC + lax.axis_index("c")
    base = wid * _BPW

    pltpu.sync_copy(meansf_hbm, means_v)
    pltpu.sync_copy(lvf_hbm, iv_v)
    pltpu.sync_copy(lvtf_hbm, lvt_v)
    pltpu.sync_copy(logc_hbm, a_v)
    pltpu.sync_copy(logb_hbm, logb_v)
    pltpu.sync_copy(ct_hbm.at[pl.ds(base, _BPW)], ct_v)
    pltpu.sync_copy(bt_hbm.at[pl.ds(base, _BPW)], bt_v)

    nk = _CP // _L
    zero = jnp.zeros((_L,), jnp.float32)

    def exp_body(i, _):
        for j in range(8):
            sl = pl.ds(i * _D + j * _L, _L)
            iv_v[sl] = jnp.exp(-iv_v[sl])
        return 0

    lax.fori_loop(0, _C, exp_body, 0)

    def sum_body(d, accs):
        return tuple(accs[k] + lvt_v[pl.ds(d * _CP + k * _L, _L)]
                     for k in range(nk))

    sums = lax.fori_loop(0, _D, sum_body, (zero,) * nk)
    for k in range(nk):
        a_v[pl.ds(k * _L, _L)] = a_v[pl.ds(k * _L, _L)] - 0.5 * sums[k]

    lane = lax.iota(jnp.int32, _L)

    for s in range(_NSUB):
        off = base * _D + s * _SUB * _D
        pltpu.sync_copy(z0f_hbm.at[pl.ds(off, _SUB * _D)], z0_v)
        pltpu.sync_copy(zTf_hbm.at[pl.ds(off, _SUB * _D)], zT_v)
        for g in range(_NGRP):
            lo = s * _SUB + g * _L
            ct = ct_v[pl.ds(lo, _L)]
            bt = bt_v[pl.ds(lo, _L)]
            cat = (plsc.load_gather(a_v, [ct])
                   + plsc.load_gather(logb_v, [bt]))
            pv0 = (jnp.int32(g * _L) + lane) * _D + lane
            bound = (jnp.int32(g * _L) + lane) * _D + _D
            mi0 = ct * _D + lane

            def dbody(_, carry):
                pv, mi, q0, q1, t0, t1 = carry
                qs = [q0, q1]
                ts = [t0, t1]
                for u in range(_UNROLL):
                    z0 = plsc.load_gather(z0_v, [pv])
                    zT = plsc.load_gather(zT_v, [pv])
                    mu = plsc.load_gather(means_v, [mi])
                    iv = plsc.load_gather(iv_v, [mi])
                    dlt = z0 - mu
                    qs[u % 2] = qs[u % 2] + dlt * dlt * iv
                    ts[u % 2] = ts[u % 2] + zT * zT
                    pv1 = pv + 1
                    adj = jnp.where(pv1 == bound, jnp.int32(_D),
                                    jnp.int32(0))
                    pv = pv1 - adj
                    mi = (mi + 1) - adj
                return (pv, mi, qs[0], qs[1], ts[0], ts[1])

            _, _, q0, q1, t0, t1 = lax.fori_loop(
                0, _D // _UNROLL, dbody,
                (pv0, mi0, zero, zero, zero, zero))
            out_v[pl.ds(lo, _L)] = (-0.5 * ((q0 + q1) + (t0 + t1))
                                    + (cat + jnp.float32(_K)))

    pltpu.sync_copy(out_v, out_hbm.at[pl.ds(base, _BPW)])


_sc_call = pl.kernel(
    _body,
    out_type=jax.ShapeDtypeStruct((_B,), jnp.float32),
    mesh=plsc.VectorSubcoreMesh(core_axis_name="c", subcore_axis_name="s"),
    compiler_params=pltpu.CompilerParams(needs_layout_passes=False),
    scratch_types=[
        pltpu.VMEM((_C * _D,), jnp.float32),
        pltpu.VMEM((_C * _D,), jnp.float32),
        pltpu.VMEM((_D * _CP,), jnp.float32),
        pltpu.VMEM((_CP,), jnp.float32),
        pltpu.VMEM((_NB,), jnp.float32),
        pltpu.VMEM((_BPW,), jnp.int32),
        pltpu.VMEM((_BPW,), jnp.int32),
        pltpu.VMEM((_SUB * _D,), jnp.float32),
        pltpu.VMEM((_SUB * _D,), jnp.float32),
        pltpu.VMEM((_BPW,), jnp.float32),
    ],
)


def kernel(z0, zT, means, log_vars, cell_probs, batch_probs,
           cell_type, batch_idx):
    z0f = z0.reshape(-1)
    zTf = zT.reshape(-1)
    meansf = means.reshape(-1)
    lvf = log_vars.reshape(-1)
    lvtf = jnp.concatenate(
        [log_vars, jnp.zeros((_CP - _C, _D), jnp.float32)]).T.reshape(-1)
    logc = jnp.log(jnp.concatenate(
        [cell_probs, jnp.ones((_CP - _C,), jnp.float32)]))
    logb = jnp.log(batch_probs)
    ct = cell_type.astype(jnp.int32)
    bt = batch_idx.astype(jnp.int32)
    return _sc_call(z0f, zTf, meansf, lvf, lvtf, logc, logb, ct, bt)


# speedup vs baseline: 6.9042x; 1.2683x over previous
import functools
import math

import jax
import jax.numpy as jnp
from jax import lax
from jax.experimental import pallas as pl
from jax.experimental.pallas import tpu as pltpu
from jax.experimental.pallas import tpu_sc as plsc

_B = 16384
_D = 128
_C = 100
_NB = 64
_CP = 112
_L = 16
_NC, _NS = 2, 16
_NW = _NC * _NS
_BPW = _B // _NW
_CH = 128
_NCH = _BPW // _CH
_GPC = _CH // _L
_P1 = _D - _L
_UNROLL = 8
_K = -_D * math.log(2.0 * math.pi)


def _body(z0f_hbm, zTf_hbm, meansi_hbm, lvf_hbm, lvtf_hbm, logc_hbm,
          logb_hbm, ct_hbm, bt_hbm, out_hbm,
          pk_v, lv_v, lvt_v, a_v, logb_v, ct_v, bt_v,
          z0b0, z0b1, zTb0, zTb1, out_v, sem0, sem1):
    wid = lax.axis_index("s") * _NC + lax.axis_index("c")
    base = wid * _BPW

    bufs = ((z0b0, zTb0, sem0), (z0b1, zTb1, sem1))

    def start_chunk(c, slot):
        off = (base + c * _CH) * _D
        z0b, zTb, sem = bufs[slot]
        h1 = pltpu.async_copy(z0f_hbm.at[pl.ds(off, _CH * _D)], z0b, sem)
        h2 = pltpu.async_copy(zTf_hbm.at[pl.ds(off, _CH * _D)], zTb, sem)
        return (h1, h2)

    handles = [start_chunk(0, 0), None]

    pltpu.sync_copy(meansi_hbm, pk_v)
    pltpu.sync_copy(lvf_hbm, lv_v)
    pltpu.sync_copy(lvtf_hbm, lvt_v)
    pltpu.sync_copy(logc_hbm, a_v)
    pltpu.sync_copy(logb_hbm, logb_v)
    pltpu.sync_copy(ct_hbm.at[pl.ds(base, _BPW)], ct_v)
    pltpu.sync_copy(bt_hbm.at[pl.ds(base, _BPW)], bt_v)

    nk = _CP // _L
    zero = jnp.zeros((_L,), jnp.float32)

    def pack_body(c, _):
        for j in range(_D // _L):
            sl = pl.ds(c * _D + j * _L, _L)
            m = plsc.bitcast(pk_v[sl], jnp.float32)
            iv = jnp.exp(-lv_v[sl])
            pk_v[sl] = plsc.bitcast(
                plsc.pack(m, iv, format=plsc.PackFormat.INTERLEAVED),
                jnp.int32)
        return 0

    lax.fori_loop(0, _C, pack_body, 0)

    def sum_body(d, accs):
        return tuple(accs[k] + lvt_v[pl.ds(d * _CP + k * _L, _L)]
                     for k in range(nk))

    sums = lax.fori_loop(0, _D, sum_body, (zero,) * nk)
    for k in range(nk):
        a_v[pl.ds(k * _L, _L)] = a_v[pl.ds(k * _L, _L)] - 0.5 * sums[k]

    lane = lax.iota(jnp.int32, _L)

    def gstep(z0b, zTb, pv, mi, qs, ts, u):
        zg = plsc.load_gather(z0b, [pv])
        tg = plsc.load_gather(zTb, [pv])
        w = plsc.load_gather(pk_v, [mi])
        mu, iv = plsc.unpack(plsc.bitcast(w, jnp.bfloat16),
                             format=plsc.PackFormat.INTERLEAVED)
        dlt = zg - mu
        qs[u % 2] = qs[u % 2] + dlt * dlt * iv
        ts[u % 2] = ts[u % 2] + tg * tg
        return qs, ts

    for c in range(_NCH):
        slot = c % 2
        if c + 1 < _NCH:
            handles[(c + 1) % 2] = start_chunk(c + 1, (c + 1) % 2)
        h1, h2 = handles[slot]
        h1.wait()
        h2.wait()
        z0b, zTb, _ = bufs[slot]

        def group_body(g, _, z0b=z0b, zTb=zTb, c=c):
            lo = c * _CH + g * _L
            ct = ct_v[pl.ds(lo, _L)]
            bt = bt_v[pl.ds(lo, _L)]
            cat = (plsc.load_gather(a_v, [ct])
                   + plsc.load_gather(logb_v, [bt]))
            sl_vec = g * _L + lane
            pv0 = sl_vec * _D + lane
            bound = sl_vec * _D + _D
            mi0 = ct * _D + lane

            def dbody1(_, carry):
                pv, mi, q0, q1, t0, t1 = carry
                qs, ts = [q0, q1], [t0, t1]
                for u in range(_UNROLL):
                    qs, ts = gstep(z0b, zTb, pv, mi, qs, ts, u)
                    pv = pv + 1
                    mi = mi + 1
                return (pv, mi, qs[0], qs[1], ts[0], ts[1])

            carry = lax.fori_loop(0, _P1 // _UNROLL, dbody1,
                                  (pv0, mi0, zero, zero, zero, zero))

            def dbody2(_, carry):
                pv, mi, q0, q1, t0, t1 = carry
                qs, ts = [q0, q1], [t0, t1]
                qs, ts = gstep(z0b, zTb, pv, mi, qs, ts, 0)
                pv1 = pv + 1
                adj = jnp.where(pv1 == bound, jnp.int32(_D), jnp.int32(0))
                return (pv1 - adj, (mi + 1) - adj,
                        qs[0], qs[1], ts[0], ts[1])

            _, _, q0, q1, t0, t1 = lax.fori_loop(0, _D - _P1, dbody2, carry)
            out_v[pl.ds(lo, _L)] = (-0.5 * ((q0 + q1) + (t0 + t1))
                                    + (cat + jnp.float32(_K)))
            return 0

        lax.fori_loop(0, _GPC, group_body, 0)

    pltpu.sync_copy(out_v, out_hbm.at[pl.ds(base, _BPW)])


_sc_call = pl.kernel(
    _body,
    out_type=jax.ShapeDtypeStruct((_B,), jnp.float32),
    mesh=plsc.VectorSubcoreMesh(core_axis_name="c", subcore_axis_name="s"),
    compiler_params=pltpu.CompilerParams(needs_layout_passes=False),
    scratch_types=[
        pltpu.VMEM((_C * _D,), jnp.int32),
        pltpu.VMEM((_C * _D,), jnp.float32),
        pltpu.VMEM((_D * _CP,), jnp.float32),
        pltpu.VMEM((_CP,), jnp.float32),
        pltpu.VMEM((_NB,), jnp.float32),
        pltpu.VMEM((_BPW,), jnp.int32),
        pltpu.VMEM((_BPW,), jnp.int32),
        pltpu.VMEM((_CH * _D,), jnp.float32),
        pltpu.VMEM((_CH * _D,), jnp.float32),
        pltpu.VMEM((_CH * _D,), jnp.float32),
        pltpu.VMEM((_CH * _D,), jnp.float32),
        pltpu.VMEM((_BPW,), jnp.float32),
        pltpu.SemaphoreType.DMA,
        pltpu.SemaphoreType.DMA,
    ],
)


def kernel(z0, zT, means, log_vars, cell_probs, batch_probs,
           cell_type, batch_idx):
    z0f = z0.reshape(-1)
    zTf = zT.reshape(-1)
    meansi = lax.bitcast_convert_type(means, jnp.int32).reshape(-1)
    lvf = log_vars.reshape(-1)
    lvtf = jnp.concatenate(
        [log_vars, jnp.zeros((_CP - _C, _D), jnp.float32)]).T.reshape(-1)
    logc = jnp.log(jnp.concatenate(
        [cell_probs, jnp.ones((_CP - _C,), jnp.float32)]))
    logb = jnp.log(batch_probs)
    ct = cell_type.astype(jnp.int32)
    bt = batch_idx.astype(jnp.int32)
    return _sc_call(z0f, zTf, meansi, lvf, lvtf, logc, logb, ct, bt)


# speedup vs baseline: 7.1036x; 1.0289x over previous
import functools
import math

import jax
import jax.numpy as jnp
from jax import lax
from jax.experimental import pallas as pl
from jax.experimental.pallas import tpu as pltpu
from jax.experimental.pallas import tpu_sc as plsc

_B = 16384
_D = 128
_C = 100
_NB = 64
_CP = 112
_L = 16
_NC, _NS = 2, 16
_NW = _NC * _NS
_BPW = _B // _NW
_CH = 128
_NCH = _BPW // _CH
_GPC = _CH // _L
_P1 = _D - _L
_NACC = 4
_K = -_D * math.log(2.0 * math.pi)


def _body(z0f_hbm, zTf_hbm, meansf_hbm, lvf_hbm, logcb_hbm,
          ct_hbm, bt_hbm, out_hbm,
          mf_v, lv_v, pk_v, a_v, ct_v, bt_v,
          z0b0, z0b1, zTb0, zTb1, out_v, sem0, sem1, semt):
    wid = lax.axis_index("s") * _NC + lax.axis_index("c")
    base = wid * _BPW

    bufs = ((z0b0, zTb0, sem0), (z0b1, zTb1, sem1))

    def start_chunk(c, slot):
        off = (base + c * _CH) * _D
        z0b, zTb, sem = bufs[slot]
        h1 = pltpu.async_copy(z0f_hbm.at[pl.ds(off, _CH * _D)], z0b, sem)
        h2 = pltpu.async_copy(zTf_hbm.at[pl.ds(off, _CH * _D)], zTb, sem)
        return (h1, h2)

    handles = [start_chunk(0, 0), None]

    th = [pltpu.async_copy(meansf_hbm, mf_v, semt),
          pltpu.async_copy(lvf_hbm, lv_v, semt),
          pltpu.async_copy(logcb_hbm, a_v, semt),
          pltpu.async_copy(ct_hbm.at[pl.ds(base, _BPW)], ct_v, semt),
          pltpu.async_copy(bt_hbm.at[pl.ds(base, _BPW)], bt_v, semt)]
    for h in th:
        h.wait()

    lane = lax.iota(jnp.int32, _L)
    zero = jnp.zeros((_L,), jnp.float32)

    def pack_body(c, _):
        for j in range(_D // _L):
            sl = pl.ds(c * _D + j * _L, _L)
            pk_v[sl] = plsc.bitcast(
                plsc.pack(mf_v[sl], jnp.exp(-lv_v[sl]),
                          format=plsc.PackFormat.INTERLEAVED),
                jnp.int32)
        return 0

    lax.fori_loop(0, _C, pack_body, 0)

    nk = _CP // _L
    cbase = tuple((k * _L + lane) * _D for k in range(nk))

    def sum_body(d, carry):
        dl = carry[0]
        accs = [carry[1 + k] + plsc.load_gather(lv_v, [cbase[k] + dl])
                for k in range(nk)]
        return ((dl + 1) & (_D - 1), *accs)

    sums = lax.fori_loop(0, _D, sum_body, (lane,) + (zero,) * nk)
    for k in range(nk):
        sl = pl.ds(k * _L, _L)
        a_v[sl] = a_v[sl] - 0.5 * sums[1 + k]

    for c in range(_NCH):
        slot = c % 2
        if c + 1 < _NCH:
            handles[(c + 1) % 2] = start_chunk(c + 1, (c + 1) % 2)
        h1, h2 = handles[slot]
        h1.wait()
        h2.wait()
        z0b, zTb, _ = bufs[slot]

        def group_body(g, _, z0b=z0b, zTb=zTb, c=c):
            lo = c * _CH + g * _L
            ct = ct_v[pl.ds(lo, _L)]
            bt = bt_v[pl.ds(lo, _L)]
            cat = (plsc.load_gather(a_v, [ct])
                   + plsc.load_gather(a_v, [bt + _CP]))
            pv0 = (g * _L + lane) * _D + lane
            mi0 = ct * _D + lane
            qs = [zero] * _NACC
            ts = [zero] * _NACC

            def gstep(pv, mi, j):
                zg = plsc.load_gather(z0b, [pv])
                tg = plsc.load_gather(zTb, [pv])
                w = plsc.load_gather(pk_v, [mi])
                mu, iv = plsc.unpack(plsc.bitcast(w, jnp.bfloat16),
                                     format=plsc.PackFormat.INTERLEAVED)
                dlt = zg - mu
                a = j % _NACC
                qs[a] = qs[a] + dlt * dlt * iv
                ts[a] = ts[a] + tg * tg

            for j in range(_P1):
                gstep(pv0 + j, mi0 + j, j)
            for j in range(_D - _P1):
                adj = jnp.where(lane >= _L - j, jnp.int32(_D), jnp.int32(0))
                gstep(pv0 + (_P1 + j) - adj, mi0 + (_P1 + j) - adj,
                      _P1 + j)

            q = (qs[0] + qs[1]) + (qs[2] + qs[3])
            t = (ts[0] + ts[1]) + (ts[2] + ts[3])
            out_v[pl.ds(lo, _L)] = (-0.5 * (q + t)
                                    + (cat + jnp.float32(_K)))
            return 0

        lax.fori_loop(0, _GPC, group_body, 0)

    pltpu.sync_copy(out_v, out_hbm.at[pl.ds(base, _BPW)])


_sc_call = pl.kernel(
    _body,
    out_type=jax.ShapeDtypeStruct((_B,), jnp.float32),
    mesh=plsc.VectorSubcoreMesh(core_axis_name="c", subcore_axis_name="s"),
    compiler_params=pltpu.CompilerParams(needs_layout_passes=False),
    scratch_types=[
        pltpu.VMEM((_C * _D,), jnp.float32),
        pltpu.VMEM((_C * _D,), jnp.float32),
        pltpu.VMEM((_C * _D,), jnp.int32),
        pltpu.VMEM((_CP + _NB,), jnp.float32),
        pltpu.VMEM((_BPW,), jnp.int32),
        pltpu.VMEM((_BPW,), jnp.int32),
        pltpu.VMEM((_CH * _D,), jnp.float32),
        pltpu.VMEM((_CH * _D,), jnp.float32),
        pltpu.VMEM((_CH * _D,), jnp.float32),
        pltpu.VMEM((_CH * _D,), jnp.float32),
        pltpu.VMEM((_BPW,), jnp.float32),
        pltpu.SemaphoreType.DMA,
        pltpu.SemaphoreType.DMA,
        pltpu.SemaphoreType.DMA,
    ],
)


def kernel(z0, zT, means, log_vars, cell_probs, batch_probs,
           cell_type, batch_idx):
    z0f = z0.reshape(-1)
    zTf = zT.reshape(-1)
    meansf = means.reshape(-1)
    lvf = log_vars.reshape(-1)
    logcb = jnp.log(jnp.concatenate(
        [cell_probs, jnp.ones((_CP - _C,), jnp.float32), batch_probs]))
    ct = cell_type.astype(jnp.int32)
    bt = batch_idx.astype(jnp.int32)
    return _sc_call(z0f, zTf, meansf, lvf, logcb, ct, bt)
